# baseline (speedup 1.0000x reference)
import jax
import jax.numpy as jnp
from jax.experimental import pallas as pl
from jax.experimental.pallas import tpu as pltpu

_B_BLOCK = 16384


def _conv_kernel(x_ref, w_ref, b_ref, o_ref):
    xg = x_ref[0].astype(jnp.bfloat16)
    o_ref[0] = (
        jnp.dot(w_ref[0], xg, preferred_element_type=jnp.float32)
        + b_ref[0, 0][:, None]
    )


def kernel(x, W, bias):
    B, G, CIN = x.shape
    COUT = W.shape[2]

    xT = jnp.transpose(x, (1, 2, 0))
    WT = jnp.transpose(W, (0, 2, 1)).astype(jnp.bfloat16)

    nb = B // _B_BLOCK
    outT = pl.pallas_call(
        _conv_kernel,
        grid=(G, nb),
        in_specs=[
            pl.BlockSpec((1, CIN, _B_BLOCK), lambda g, i: (g, 0, i)),
            pl.BlockSpec((1, COUT, CIN), lambda g, i: (g, 0, 0)),
            pl.BlockSpec((1, 1, COUT), lambda g, i: (g, 0, 0)),
        ],
        out_specs=pl.BlockSpec((1, COUT, _B_BLOCK), lambda g, i: (g, 0, i)),
        out_shape=jax.ShapeDtypeStruct((G, COUT, B), x.dtype),
        compiler_params=pltpu.CompilerParams(
            dimension_semantics=("parallel", "parallel"),
        ),
    )(xT, WT, bias.reshape(G, 1, COUT))

    return jnp.transpose(outT, (2, 0, 1))


# speedup vs baseline: 1.0323x; 1.0323x over previous
import jax
import jax.numpy as jnp
from jax.experimental import pallas as pl
from jax.experimental.pallas import tpu as pltpu


def _conv_kernel(x_ref, w_ref, b_ref, o_ref):
    xg = x_ref[0].astype(jnp.bfloat16)
    wg = w_ref[0].astype(jnp.bfloat16)
    acc = jax.lax.dot_general(
        wg, xg, (((0,), (0,)), ((), ())),
        preferred_element_type=jnp.float32,
    )
    o_ref[0] = acc + b_ref[0, 0][:, None]


def kernel(x, W, bias):
    B, G, CIN = x.shape
    COUT = W.shape[2]

    xT = jnp.transpose(x, (1, 2, 0))

    outT = pl.pallas_call(
        _conv_kernel,
        grid=(G,),
        in_specs=[
            pl.BlockSpec((1, CIN, B), lambda g: (g, 0, 0)),
            pl.BlockSpec((1, CIN, COUT), lambda g: (g, 0, 0)),
            pl.BlockSpec((1, 1, COUT), lambda g: (g, 0, 0)),
        ],
        out_specs=pl.BlockSpec((1, COUT, B), lambda g: (g, 0, 0)),
        out_shape=jax.ShapeDtypeStruct((G, COUT, B), x.dtype),
        compiler_params=pltpu.CompilerParams(
            dimension_semantics=("parallel",),
        ),
    )(xT, W, bias.reshape(G, 1, COUT))

    return jnp.transpose(outT, (2, 0, 1))
